# baseline (device time: 108333 ns/iter reference)
import functools

import jax
import jax.numpy as jnp
from jax import lax
from jax.experimental import pallas as pl
from jax.experimental.pallas import tpu as pltpu

N_DEV = 8

_CompilerParams = getattr(pltpu, "CompilerParams", None) or getattr(
    pltpu, "TPUCompilerParams"
)


def kernel(x, router_W, route_idx, expert_W):
    n_tok, d = x.shape
    n_exp = router_W.shape[1]
    e_loc, _, h = expert_W.shape

    def body(x_ref, rw_ref, idx_ref, ew_ref, out_ref,
             xg, wg, ewb, rsbuf, sendbuf,
             sx, rx, sw, rw_sem, ss, rs_sem):
        l = lax.axis_index("i")
        vp = jnp.where(l < 4, l, 11 - l)
        vp_r = (vp + 1) % N_DEV
        vp_l = (vp + N_DEV - 1) % N_DEV
        l_r = jnp.where(vp_r < 4, vp_r, 11 - vp_r)
        l_l = jnp.where(vp_l < 4, vp_l, 11 - vp_l)

        bsem = pltpu.get_barrier_semaphore()
        pl.semaphore_signal(bsem, inc=1, device_id=(l_l,),
                            device_id_type=pl.DeviceIdType.MESH)
        pl.semaphore_signal(bsem, inc=1, device_id=(l_r,),
                            device_id_type=pl.DeviceIdType.MESH)
        pl.semaphore_wait(bsem, 2)

        xf = x_ref[:, :]
        scores = jnp.dot(xf, rw_ref[:, :], preferred_element_type=jnp.float32)
        m = jnp.max(scores, axis=1, keepdims=True)
        p = jnp.exp(scores - m)
        p = p / jnp.sum(p, axis=1, keepdims=True)
        e_iota = lax.broadcasted_iota(jnp.int32, (n_tok, n_exp), 1)
        oh0 = (e_iota == idx_ref[:, 0:1]).astype(jnp.float32)
        oh1 = (e_iota == idx_ref[:, 1:2]).astype(jnp.float32)
        g0 = jnp.sum(p * oh0, axis=1, keepdims=True)
        g1 = jnp.sum(p * oh1, axis=1, keepdims=True)
        wmat = (oh0 * g0 + oh1 * g1) / (g0 + g1)

        xg[vp] = xf.astype(jnp.bfloat16)
        wg[vp] = wmat.astype(jnp.bfloat16)
        for e in range(e_loc):
            ewb[e] = ew_ref[e].astype(jnp.bfloat16)

        for hh in range(N_DEV - 1):
            src = (vp - hh) % N_DEV
            rd_x = pltpu.make_async_remote_copy(
                src_ref=xg.at[src], dst_ref=xg.at[src],
                send_sem=sx.at[hh], recv_sem=rx.at[hh],
                device_id=(l_r,), device_id_type=pl.DeviceIdType.MESH)
            rd_w = pltpu.make_async_remote_copy(
                src_ref=wg.at[src], dst_ref=wg.at[src],
                send_sem=sw.at[hh], recv_sem=rw_sem.at[hh],
                device_id=(l_r,), device_id_type=pl.DeviceIdType.MESH)
            rd_x.start()
            rd_w.start()
            rd_x.wait()
            rd_w.wait()

        def partial(c):
            xc = xg[c]
            wc = wg[c].astype(jnp.float32)
            acc = jnp.zeros((n_tok, h), jnp.float32)
            for e in range(e_loc):
                col = e_loc * l + e
                gate = jnp.sum(
                    wc * (e_iota == col).astype(jnp.float32),
                    axis=1, keepdims=True)
                acc = acc + jnp.dot(
                    xc, ewb[e], preferred_element_type=jnp.float32) * gate
            return acc

        for s in range(N_DEV - 1):
            c_send = (vp - 1 - s) % N_DEV
            acc = partial(c_send)
            if s > 0:
                acc = acc + rsbuf[s - 1].astype(jnp.float32)
            sendbuf[s % 2] = acc.astype(jnp.bfloat16)
            rd = pltpu.make_async_remote_copy(
                src_ref=sendbuf.at[s % 2], dst_ref=rsbuf.at[s],
                send_sem=ss.at[s], recv_sem=rs_sem.at[s],
                device_id=(l_r,), device_id_type=pl.DeviceIdType.MESH)
            rd.start()
            rd.wait()

        out_ref[:, :] = partial(vp) + rsbuf[N_DEV - 2].astype(jnp.float32)

    return pl.pallas_call(
        body,
        out_shape=jax.ShapeDtypeStruct((n_tok, h), jnp.float32),
        in_specs=[
            pl.BlockSpec(memory_space=pltpu.VMEM),
            pl.BlockSpec(memory_space=pltpu.VMEM),
            pl.BlockSpec(memory_space=pltpu.VMEM),
            pl.BlockSpec(memory_space=pltpu.VMEM),
        ],
        out_specs=pl.BlockSpec(memory_space=pltpu.VMEM),
        scratch_shapes=[
            pltpu.VMEM((N_DEV, n_tok, d), jnp.bfloat16),
            pltpu.VMEM((N_DEV, n_tok, n_exp), jnp.bfloat16),
            pltpu.VMEM((e_loc, d, h), jnp.bfloat16),
            pltpu.VMEM((N_DEV - 1, n_tok, h), jnp.bfloat16),
            pltpu.VMEM((2, n_tok, h), jnp.bfloat16),
            pltpu.SemaphoreType.DMA((N_DEV - 1,)),
            pltpu.SemaphoreType.DMA((N_DEV - 1,)),
            pltpu.SemaphoreType.DMA((N_DEV - 1,)),
            pltpu.SemaphoreType.DMA((N_DEV - 1,)),
            pltpu.SemaphoreType.DMA((N_DEV - 1,)),
            pltpu.SemaphoreType.DMA((N_DEV - 1,)),
        ],
        compiler_params=_CompilerParams(collective_id=0),
    )(x, router_W, route_idx, expert_W)


# device time: 98293 ns/iter; 1.1021x vs baseline; 1.1021x over previous
import functools

import jax
import jax.numpy as jnp
from jax import lax
from jax.experimental import pallas as pl
from jax.experimental.pallas import tpu as pltpu

N_DEV = 8

_CompilerParams = getattr(pltpu, "CompilerParams", None) or getattr(
    pltpu, "TPUCompilerParams"
)


def kernel(x, router_W, route_idx, expert_W):
    n_tok, d = x.shape
    n_exp = router_W.shape[1]
    e_loc, _, h = expert_W.shape

    def body(x_ref, rw_ref, idx_ref, ew_ref, out_ref,
             xg, wg, ewb, rsbuf, sendbuf,
             sx, rx, sw, rw_sem, ss, rs_sem):
        l = lax.axis_index("i")
        vp = jnp.where(l < 4, l, 11 - l)
        vp_r = (vp + 1) % N_DEV
        vp_l = (vp + N_DEV - 1) % N_DEV
        l_r = jnp.where(vp_r < 4, vp_r, 11 - vp_r)
        l_l = jnp.where(vp_l < 4, vp_l, 11 - vp_l)

        bsem = pltpu.get_barrier_semaphore()
        pl.semaphore_signal(bsem, inc=1, device_id=(l_l,),
                            device_id_type=pl.DeviceIdType.MESH)
        pl.semaphore_signal(bsem, inc=1, device_id=(l_r,),
                            device_id_type=pl.DeviceIdType.MESH)
        pl.semaphore_wait(bsem, 2)

        xf = x_ref[:, :]
        scores = jnp.dot(xf, rw_ref[:, :], preferred_element_type=jnp.float32)
        m = jnp.max(scores, axis=1, keepdims=True)
        p = jnp.exp(scores - m)
        p = p / jnp.sum(p, axis=1, keepdims=True)
        e_iota = lax.broadcasted_iota(jnp.int32, (n_tok, n_exp), 1)
        oh0 = (e_iota == idx_ref[:, 0:1]).astype(jnp.float32)
        oh1 = (e_iota == idx_ref[:, 1:2]).astype(jnp.float32)
        g0 = jnp.sum(p * oh0, axis=1, keepdims=True)
        g1 = jnp.sum(p * oh1, axis=1, keepdims=True)
        wmat = (oh0 * g0 + oh1 * g1) / (g0 + g1)

        xg[vp] = xf.astype(jnp.bfloat16)
        wg[vp] = wmat.astype(jnp.bfloat16)
        for e in range(e_loc):
            ewb[e] = ew_ref[e].astype(jnp.bfloat16)

        def mk_ag_x(hh, slot):
            return pltpu.make_async_remote_copy(
                src_ref=xg.at[slot], dst_ref=xg.at[slot],
                send_sem=sx.at[hh], recv_sem=rx.at[hh],
                device_id=(l_r,), device_id_type=pl.DeviceIdType.MESH)

        def mk_ag_w(hh, slot):
            return pltpu.make_async_remote_copy(
                src_ref=wg.at[slot], dst_ref=wg.at[slot],
                send_sem=sw.at[hh], recv_sem=rw_sem.at[hh],
                device_id=(l_r,), device_id_type=pl.DeviceIdType.MESH)

        def mk_rs(hh):
            return pltpu.make_async_remote_copy(
                src_ref=sendbuf.at[hh % 2], dst_ref=rsbuf.at[hh],
                send_sem=ss.at[hh], recv_sem=rs_sem.at[hh],
                device_id=(l_r,), device_id_type=pl.DeviceIdType.MESH)

        def partial(c):
            xc = xg[c]
            wc = wg[c].astype(jnp.float32)
            acc = jnp.zeros((n_tok, h), jnp.float32)
            for e in range(e_loc):
                col = e_loc * l + e
                gate = jnp.sum(
                    wc * (e_iota == col).astype(jnp.float32),
                    axis=1, keepdims=True)
                acc = acc + jnp.dot(
                    xc, ewb[e], preferred_element_type=jnp.float32) * gate
            return acc

        for s in range(N_DEV - 1):
            src = (vp - s) % N_DEV
            agx = mk_ag_x(s, src)
            agw = mk_ag_w(s, src)
            agx.start()
            agw.start()
            if s >= 2:
                mk_rs(s - 2).wait_send()
            dst = (vp - 1 - s) % N_DEV
            mk_ag_x(s, dst).wait_recv()
            mk_ag_w(s, dst).wait_recv()
            acc = partial(dst)
            if s >= 1:
                mk_rs(s - 1).wait_recv()
                acc = acc + rsbuf[s - 1].astype(jnp.float32)
            sendbuf[s % 2] = acc.astype(jnp.bfloat16)
            mk_rs(s).start()

        accv = partial(vp)
        mk_rs(N_DEV - 2).wait_recv()
        out_ref[:, :] = accv + rsbuf[N_DEV - 2].astype(jnp.float32)

        for s in range(N_DEV - 1):
            mk_ag_x(s, (vp - s) % N_DEV).wait_send()
            mk_ag_w(s, (vp - s) % N_DEV).wait_send()
        mk_rs(N_DEV - 3).wait_send()
        mk_rs(N_DEV - 2).wait_send()

    return pl.pallas_call(
        body,
        out_shape=jax.ShapeDtypeStruct((n_tok, h), jnp.float32),
        in_specs=[
            pl.BlockSpec(memory_space=pltpu.VMEM),
            pl.BlockSpec(memory_space=pltpu.VMEM),
            pl.BlockSpec(memory_space=pltpu.VMEM),
            pl.BlockSpec(memory_space=pltpu.VMEM),
        ],
        out_specs=pl.BlockSpec(memory_space=pltpu.VMEM),
        scratch_shapes=[
            pltpu.VMEM((N_DEV, n_tok, d), jnp.bfloat16),
            pltpu.VMEM((N_DEV, n_tok, n_exp), jnp.bfloat16),
            pltpu.VMEM((e_loc, d, h), jnp.bfloat16),
            pltpu.VMEM((N_DEV - 1, n_tok, h), jnp.bfloat16),
            pltpu.VMEM((2, n_tok, h), jnp.bfloat16),
            pltpu.SemaphoreType.DMA((N_DEV - 1,)),
            pltpu.SemaphoreType.DMA((N_DEV - 1,)),
            pltpu.SemaphoreType.DMA((N_DEV - 1,)),
            pltpu.SemaphoreType.DMA((N_DEV - 1,)),
            pltpu.SemaphoreType.DMA((N_DEV - 1,)),
            pltpu.SemaphoreType.DMA((N_DEV - 1,)),
        ],
        compiler_params=_CompilerParams(collective_id=0),
    )(x, router_W, route_idx, expert_W)


# device time: 64940 ns/iter; 1.6682x vs baseline; 1.5136x over previous
import jax
import jax.numpy as jnp
from jax import lax
from jax.experimental import pallas as pl
from jax.experimental.pallas import tpu as pltpu

N_DEV = 8
N_R = 4
N_L = 3

_CompilerParams = getattr(pltpu, "CompilerParams", None) or getattr(
    pltpu, "TPUCompilerParams"
)


def kernel(x, router_W, route_idx, expert_W):
    n_tok, d = x.shape
    n_exp = router_W.shape[1]
    e_loc, _, h = expert_W.shape
    h2 = h // 2

    def body(x_ref, rw_ref, idx_ref, ew_ref, out_ref,
             xg, wg, ewb, rsbuf_r, rsbuf_l, sbuf_r, sbuf_l,
             sxr, rxr, swr, rwr, sxl, rxl, swl, rwl,
             ssr, rsr, ssl, rsl):
        l = lax.axis_index("i")
        vp = jnp.where(l < 4, l, 11 - l)
        vp_r = (vp + 1) % N_DEV
        vp_l = (vp + N_DEV - 1) % N_DEV
        l_r = jnp.where(vp_r < 4, vp_r, 11 - vp_r)
        l_l = jnp.where(vp_l < 4, vp_l, 11 - vp_l)

        bsem = pltpu.get_barrier_semaphore()
        pl.semaphore_signal(bsem, inc=1, device_id=(l_l,),
                            device_id_type=pl.DeviceIdType.MESH)
        pl.semaphore_signal(bsem, inc=1, device_id=(l_r,),
                            device_id_type=pl.DeviceIdType.MESH)
        pl.semaphore_wait(bsem, 2)

        xf = x_ref[:, :]
        scores = jnp.dot(xf, rw_ref[:, :], preferred_element_type=jnp.float32)
        m = jnp.max(scores, axis=1, keepdims=True)
        p = jnp.exp(scores - m)
        p = p / jnp.sum(p, axis=1, keepdims=True)
        e_iota = lax.broadcasted_iota(jnp.int32, (n_tok, n_exp), 1)
        oh0 = (e_iota == idx_ref[:, 0:1]).astype(jnp.float32)
        oh1 = (e_iota == idx_ref[:, 1:2]).astype(jnp.float32)
        g0 = jnp.sum(p * oh0, axis=1, keepdims=True)
        g1 = jnp.sum(p * oh1, axis=1, keepdims=True)
        wmat = (oh0 * g0 + oh1 * g1) / (g0 + g1)

        xg[vp] = xf.astype(jnp.bfloat16)
        wg[vp] = wmat.astype(jnp.bfloat16)
        for e in range(e_loc):
            ewb[e] = ew_ref[e].astype(jnp.bfloat16)

        def mk_ag(hh, slot, to_right, is_x):
            buf, sems = (
                (xg, (sxr, rxr) if to_right else (sxl, rxl)) if is_x
                else (wg, (swr, rwr) if to_right else (swl, rwl))
            )
            return pltpu.make_async_remote_copy(
                src_ref=buf.at[slot], dst_ref=buf.at[slot],
                send_sem=sems[0].at[hh], recv_sem=sems[1].at[hh],
                device_id=(l_r if to_right else l_l,),
                device_id_type=pl.DeviceIdType.MESH)

        def mk_rs(hh, to_right):
            sb, rb, sems = (
                (sbuf_r, rsbuf_r, (ssr, rsr)) if to_right
                else (sbuf_l, rsbuf_l, (ssl, rsl))
            )
            return pltpu.make_async_remote_copy(
                src_ref=sb.at[hh % 2], dst_ref=rb.at[hh],
                send_sem=sems[0].at[hh], recv_sem=sems[1].at[hh],
                device_id=(l_r if to_right else l_l,),
                device_id_type=pl.DeviceIdType.MESH)

        def partial_half(c, hi):
            xc = xg[c]
            wc = wg[c].astype(jnp.float32)
            acc = jnp.zeros((n_tok, h2), jnp.float32)
            for e in range(e_loc):
                col = e_loc * l + e
                gate = jnp.sum(
                    wc * (e_iota == col).astype(jnp.float32),
                    axis=1, keepdims=True)
                we = ewb[e][:, h2:] if hi else ewb[e][:, :h2]
                acc = acc + jnp.dot(
                    xc, we, preferred_element_type=jnp.float32) * gate
            return acc

        for s in range(N_DEV - 1):
            if s < N_R:
                for is_x in (True, False):
                    mk_ag(s, (vp - s) % N_DEV, True, is_x).start()
            if s < N_L:
                for is_x in (True, False):
                    mk_ag(s, (vp + s) % N_DEV, False, is_x).start()
            if s >= 2:
                mk_rs(s - 2, True).wait_send()
                mk_rs(s - 2, False).wait_send()
            if 1 <= s <= N_R:
                for is_x in (True, False):
                    mk_ag(s - 1, (vp - s + 1) % N_DEV, True, is_x).wait_send()
            if 1 <= s <= N_L:
                for is_x in (True, False):
                    mk_ag(s - 1, (vp + s - 1) % N_DEV, False, is_x).wait_send()
            if s < N_R:
                for is_x in (True, False):
                    mk_ag(s, (vp - 1 - s) % N_DEV, True, is_x).wait_recv()
            if s < N_L:
                for is_x in (True, False):
                    mk_ag(s, (vp + 1 + s) % N_DEV, False, is_x).wait_recv()
            acc_r = partial_half((vp - 1 - s) % N_DEV, 0)
            if s >= 1:
                mk_rs(s - 1, True).wait_recv()
                acc_r = acc_r + rsbuf_r[s - 1].astype(jnp.float32)
            sbuf_r[s % 2] = acc_r.astype(jnp.bfloat16)
            mk_rs(s, True).start()
            acc_l = partial_half((vp + 1 + s) % N_DEV, 1)
            if s >= 1:
                mk_rs(s - 1, False).wait_recv()
                acc_l = acc_l + rsbuf_l[s - 1].astype(jnp.float32)
            sbuf_l[s % 2] = acc_l.astype(jnp.bfloat16)
            mk_rs(s, False).start()

        own_r = partial_half(vp, 0)
        mk_rs(N_DEV - 2, True).wait_recv()
        out_ref[:, :h2] = own_r + rsbuf_r[N_DEV - 2].astype(jnp.float32)
        own_l = partial_half(vp, 1)
        mk_rs(N_DEV - 2, False).wait_recv()
        out_ref[:, h2:] = own_l + rsbuf_l[N_DEV - 2].astype(jnp.float32)

        for to_right in (True, False):
            mk_rs(N_DEV - 3, to_right).wait_send()
            mk_rs(N_DEV - 2, to_right).wait_send()

    return pl.pallas_call(
        body,
        out_shape=jax.ShapeDtypeStruct((n_tok, h), jnp.float32),
        in_specs=[
            pl.BlockSpec(memory_space=pltpu.VMEM),
            pl.BlockSpec(memory_space=pltpu.VMEM),
            pl.BlockSpec(memory_space=pltpu.VMEM),
            pl.BlockSpec(memory_space=pltpu.VMEM),
        ],
        out_specs=pl.BlockSpec(memory_space=pltpu.VMEM),
        scratch_shapes=[
            pltpu.VMEM((N_DEV, n_tok, d), jnp.bfloat16),
            pltpu.VMEM((N_DEV, n_tok, n_exp), jnp.bfloat16),
            pltpu.VMEM((e_loc, d, h), jnp.bfloat16),
            pltpu.VMEM((N_DEV - 1, n_tok, h2), jnp.bfloat16),
            pltpu.VMEM((N_DEV - 1, n_tok, h2), jnp.bfloat16),
            pltpu.VMEM((2, n_tok, h2), jnp.bfloat16),
            pltpu.VMEM((2, n_tok, h2), jnp.bfloat16),
            pltpu.SemaphoreType.DMA((N_R,)),
            pltpu.SemaphoreType.DMA((N_R,)),
            pltpu.SemaphoreType.DMA((N_R,)),
            pltpu.SemaphoreType.DMA((N_R,)),
            pltpu.SemaphoreType.DMA((N_L,)),
            pltpu.SemaphoreType.DMA((N_L,)),
            pltpu.SemaphoreType.DMA((N_L,)),
            pltpu.SemaphoreType.DMA((N_L,)),
            pltpu.SemaphoreType.DMA((N_DEV - 1,)),
            pltpu.SemaphoreType.DMA((N_DEV - 1,)),
            pltpu.SemaphoreType.DMA((N_DEV - 1,)),
            pltpu.SemaphoreType.DMA((N_DEV - 1,)),
        ],
        compiler_params=_CompilerParams(collective_id=0),
    )(x, router_W, route_idx, expert_W)
